# baseline (speedup 1.0000x reference)
import functools

import jax
import jax.numpy as jnp
from jax import lax
from jax.experimental import pallas as pl
from jax.experimental.pallas import tpu as pltpu
from jax.experimental.pallas import tpu_sc as plsc

_VOCAB = 1000000
_D = 64
_S = 200
_B = 4096
_NW = 32
_SEQ_PER_W = _B // _NW
_CHUNK_A = 128
_CHUNK_B = _S - _CHUNK_A
_LANES = 16
_VECS_PER_SEQ = (_S * _D) // _LANES


def _positional_table():
    position = jnp.arange(0, _S, dtype=jnp.float32).reshape((_S, 1))
    even_i = jnp.arange(0, _D, 2, dtype=jnp.float32)
    odd_i = jnp.arange(1, _D, 2, dtype=jnp.float32)
    pow_even = jnp.power(10000.0, 2.0 * even_i / _D)
    pow_odd = jnp.power(10000.0, 2.0 * odd_i / _D)
    pe_even = jnp.sin(position / pow_even)
    pe_odd = jnp.cos(position / pow_odd)
    pe = jnp.stack([pe_even, pe_odd], axis=2).reshape(_S, _D)
    return pe


def _body(idx_hbm, pe_hbm, table_hbm, out_hbm, pe_v, idx_a, idx_b, rows_v, sem):
    wid = lax.axis_index("s") * 2 + lax.axis_index("c")
    pltpu.sync_copy(pe_hbm, pe_v)

    @pl.loop(0, _SEQ_PER_W)
    def _seq(g):
        b = wid * _SEQ_PER_W + g
        base = b * _S
        pltpu.sync_copy(idx_hbm.at[pl.ds(base, _CHUNK_A)], idx_a)
        pltpu.sync_copy(idx_hbm.at[pl.ds(base + _CHUNK_A, _CHUNK_B)], idx_b)
        d1 = pltpu.async_copy(table_hbm.at[idx_a], rows_v.at[pl.ds(0, _CHUNK_A)], sem)
        d2 = pltpu.async_copy(table_hbm.at[idx_b], rows_v.at[pl.ds(_CHUNK_A, _CHUNK_B)], sem)
        d1.wait()
        d2.wait()

        @pl.loop(0, _S)
        def _row(r):
            for c in range(_D // _LANES):
                sl = pl.ds(c * _LANES, _LANES)
                rows_v[r, sl] = rows_v[r, sl] + pe_v[r, sl]

        pltpu.sync_copy(rows_v, out_hbm.at[b])


@functools.partial(jax.jit, static_argnames=())
def kernel(data, table):
    idx = data.reshape(-1).astype(jnp.int32)
    pe = _positional_table()
    mesh = plsc.VectorSubcoreMesh(core_axis_name="c", subcore_axis_name="s")
    out = pl.kernel(
        _body,
        out_type=jax.ShapeDtypeStruct((_B, _S, _D), jnp.float32),
        mesh=mesh,
        scratch_types=[
            pltpu.VMEM((_S, _D), jnp.float32),
            pltpu.VMEM((_CHUNK_A,), jnp.int32),
            pltpu.VMEM((_CHUNK_B,), jnp.int32),
            pltpu.VMEM((_S, _D), jnp.float32),
            pltpu.SemaphoreType.DMA,
        ],
        compiler_params=pltpu.CompilerParams(use_tc_tiling_on_sc=False),
    )(idx, pe, table)
    return out


# speedup vs baseline: 1.1437x; 1.1437x over previous
import functools

import jax
import jax.numpy as jnp
from jax import lax
from jax.experimental import pallas as pl
from jax.experimental.pallas import tpu as pltpu
from jax.experimental.pallas import tpu_sc as plsc

_D = 64
_S = 200
_B = 4096
_NW = 32
_SEQ_PER_W = _B // _NW
_CHUNK_A = 128
_CHUNK_B = _S - _CHUNK_A
_LANES = 16
_NBUF = 4


def _positional_table():
    position = jnp.arange(0, _S, dtype=jnp.float32).reshape((_S, 1))
    even_i = jnp.arange(0, _D, 2, dtype=jnp.float32)
    odd_i = jnp.arange(1, _D, 2, dtype=jnp.float32)
    pow_even = jnp.power(10000.0, 2.0 * even_i / _D)
    pow_odd = jnp.power(10000.0, 2.0 * odd_i / _D)
    pe_even = jnp.sin(position / pow_even)
    pe_odd = jnp.cos(position / pow_odd)
    pe = jnp.stack([pe_even, pe_odd], axis=2).reshape(_S, _D)
    return pe


def _body(idx_hbm, pe_hbm, table_hbm, out_hbm, pe_v, idx2, rows2, sg, sw):
    wid = lax.axis_index("s") * 2 + lax.axis_index("c")
    base_seq = wid * _SEQ_PER_W
    pltpu.sync_copy(pe_hbm, pe_v)

    def fire_gather(seq, b):
        pltpu.sync_copy(idx_hbm.at[pl.ds((base_seq + seq) * _S, _S)], idx2.at[b])
        pltpu.async_copy(
            table_hbm.at[idx2.at[b, pl.ds(0, _CHUNK_A)]],
            rows2.at[b, pl.ds(0, _CHUNK_A)], sg[b])
        pltpu.async_copy(
            table_hbm.at[idx2.at[b, pl.ds(_CHUNK_A, _CHUNK_B)]],
            rows2.at[b, pl.ds(_CHUNK_A, _CHUNK_B)], sg[b])

    def drain_gather(b):
        pltpu.make_async_copy(
            table_hbm.at[pl.ds(0, _S)], rows2.at[b], sg[b]).wait()

    def drain_writeout(b):
        pltpu.make_async_copy(
            rows2.at[b], out_hbm.at[base_seq], sw[b]).wait()

    for b in range(_NBUF - 1):
        fire_gather(b, b)

    @pl.loop(0, _SEQ_PER_W, step=_NBUF)
    def _block(t):
        for b in range(_NBUF):
            s = t + b
            bn = (b + _NBUF - 1) % _NBUF
            @pl.when(s < _SEQ_PER_W - (_NBUF - 1))
            def _refill():
                @pl.when(s >= 1)
                def _drain_prev():
                    drain_writeout(bn)
                fire_gather(s + (_NBUF - 1), bn)

            drain_gather(b)

            @pl.loop(0, _S, unroll=4)
            def _row(r):
                for c in range(_D // _LANES):
                    sl = pl.ds(c * _LANES, _LANES)
                    plsc.addupdate(rows2.at[b, r, sl], pe_v[r, sl])

            pltpu.async_copy(rows2.at[b], out_hbm.at[base_seq + s], sw[b])

    for b in range(_NBUF):
        drain_writeout(b)


@jax.jit
def kernel(data, table):
    idx = data.reshape(-1).astype(jnp.int32)
    pe = _positional_table()
    mesh = plsc.VectorSubcoreMesh(core_axis_name="c", subcore_axis_name="s")
    out = pl.kernel(
        _body,
        out_type=jax.ShapeDtypeStruct((_B, _S, _D), jnp.float32),
        mesh=mesh,
        scratch_types=[
            pltpu.VMEM((_S, _D), jnp.float32),
            pltpu.VMEM((_NBUF, _S), jnp.int32),
            pltpu.VMEM((_NBUF, _S, _D), jnp.float32),
            [pltpu.SemaphoreType.DMA] * _NBUF,
            [pltpu.SemaphoreType.DMA] * _NBUF,
        ],
        compiler_params=pltpu.CompilerParams(use_tc_tiling_on_sc=False),
    )(idx, pe, table)
    return out
